# Initial kernel scaffold; baseline (speedup 1.0000x reference)
import jax
import jax.numpy as jnp
from jax.experimental import pallas as pl


def kernel(x, edge_index, W1, as1, ad1, b1, g1, be1, W2, as2, ad2, b2, g2, be2, Wh, bh):
    raise NotImplementedError("write your pallas kernel here")


# speedup vs baseline: 37.3083x; 37.3083x over previous
import functools

import jax
import jax.numpy as jnp
from jax import lax
from jax.experimental import pallas as pl
from jax.experimental.pallas import tpu as pltpu
from jax.experimental.pallas import tpu_sc as plsc

N = 10000
D = 128
C = 128
H = 2
E = 320000
NP = 10240
TRASH = 10200
CH = 128
NTILE = 16
NSC = 2
ET = 20736
NCHUNK = ET // CH
EPAD = NTILE * ET


def _prep_body(xin_ref, w_ref, asr_ref, adr_ref, xlp_ref, asv_ref, adv_ref, m_ref):
    xin = xin_ref[...]
    for h in range(H):
        xl = jnp.dot(xin, w_ref[:, h * C:(h + 1) * C],
                     preferred_element_type=jnp.float32)
        a_s = jnp.sum(xl * asr_ref[h, :][None, :], axis=1)
        a_d = jnp.sum(xl * adr_ref[h, :][None, :], axis=1)
        xlp_ref[h] = jnp.concatenate(
            [xl, jnp.zeros((NP - N, C), jnp.float32)], axis=0)
        asv_ref[h] = jnp.concatenate([a_s, jnp.zeros((NP - N,), jnp.float32)])
        adv_ref[h] = jnp.concatenate([a_d, jnp.zeros((NP - N,), jnp.float32)])
        mm = jnp.max(a_s) + jnp.max(a_d)
        m_ref[h] = jnp.full((16,), jnp.maximum(mm, 0.2 * mm))


def _tc_prep(xin, w, a_src, a_dst):
    return pl.pallas_call(
        _prep_body,
        out_shape=(
            jax.ShapeDtypeStruct((H, NP, C), jnp.float32),
            jax.ShapeDtypeStruct((H, NP), jnp.float32),
            jax.ShapeDtypeStruct((H, NP), jnp.float32),
            jax.ShapeDtypeStruct((H, 16), jnp.float32),
        ),
    )(xin, w, a_src, a_dst)


def _merge_heads(acc_ref, den_ref):
    o = None
    for h in range(H):
        den = jnp.sum(den_ref[h], axis=0)[:N, None] + 1e-16
        oh = acc_ref[h, :N, :] / den
        o = oh if o is None else o + oh
    return o * (1.0 / H)


def _bn_relu(hv, g, b):
    mu = jnp.mean(hv, axis=0, keepdims=True)
    var = jnp.mean((hv - mu) ** 2, axis=0, keepdims=True)
    return jnp.maximum((hv - mu) / jnp.sqrt(var + 1e-5) * g[None, :] + b[None, :],
                       0.0)


def _finish_body(acc_ref, den_ref, res_ref, b_ref, g_ref, be_ref, out_ref):
    hv = _merge_heads(acc_ref, den_ref) + b_ref[...][None, :] + res_ref[...]
    out_ref[...] = _bn_relu(hv, g_ref[...], be_ref[...])


def _head_body(acc_ref, den_ref, res_ref, b_ref, g_ref, be_ref, wh_ref, bh_ref,
               out_ref):
    hv = _merge_heads(acc_ref, den_ref) + b_ref[...][None, :] + res_ref[...]
    hv = _bn_relu(hv, g_ref[...], be_ref[...])
    out_ref[...] = jnp.dot(hv, wh_ref[...],
                           preferred_element_type=jnp.float32)[:, 0] + bh_ref[0]


def _sc_edge_body(xlp_hbm, asv_hbm, adv_hbm, m_hbm, src_hbm, dst_hbm,
                  out_hbm, den_hbm,
                  asrc_t, adst_t, m_b, src_b, dst_b, a_b, rows, denom_t,
                  acc_s, sem):
    c = lax.axis_index("c")
    s = lax.axis_index("s")
    row_off = c * NP
    pltpu.sync_copy(asv_hbm.at[pl.ds(row_off, NP)], asrc_t)
    pltpu.sync_copy(adv_hbm.at[pl.ds(row_off, NP)], adst_t)
    pltpu.sync_copy(m_hbm.at[pl.ds(c * 16, 16)], m_b)

    def zden(i, carry):
        denom_t[pl.ds(i * 16, 16)] = jnp.zeros((16,), jnp.float32)
        return carry
    lax.fori_loop(0, NP // 16, zden, 0)

    def zrow(r, carry):
        for j in range(C // 16):
            rows[r, pl.ds(j * 16, 16)] = jnp.zeros((16,), jnp.float32)
        return carry
    lax.fori_loop(0, CH, zrow, 0)
    base = s * (NP // NTILE)
    for k in range(NP // NTILE // CH):
        pltpu.sync_copy(rows, acc_s.at[pl.ds(base + k * CH, CH)])
    plsc.subcore_barrier()

    m_vec = m_b[...]
    e0 = s * ET

    def chunk(g, carry):
        off = e0 + g * CH
        pltpu.sync_copy(src_hbm.at[pl.ds(off, CH)], src_b)
        pltpu.sync_copy(dst_hbm.at[pl.ds(off, CH)], dst_b)

        def avec(i, cc):
            sv = src_b[pl.ds(i * 16, 16)]
            dv = dst_b[pl.ds(i * 16, 16)]
            al = plsc.load_gather(asrc_t, [sv]) + plsc.load_gather(adst_t, [dv])
            al = jnp.maximum(al, 0.2 * al)
            ae = jnp.exp(al - m_vec)
            a_b[pl.ds(i * 16, 16)] = ae
            plsc.addupdate_scatter(denom_t, [dv], ae)
            src_b[pl.ds(i * 16, 16)] = sv + row_off
            return cc
        lax.fori_loop(0, CH // 16, avec, 0)

        pltpu.async_copy(xlp_hbm.at[src_b], rows, sem).wait()

        def srow(i, cc):
            av = a_b[pl.ds(i * 16, 16)]
            for l in range(16):
                svec = jnp.full((16,), av[l], jnp.float32)
                e = i * 16 + l
                for j in range(C // 16):
                    rows[e, pl.ds(j * 16, 16)] = rows[e, pl.ds(j * 16, 16)] * svec
            return cc
        lax.fori_loop(0, CH // 16, srow, 0)

        pltpu.sync_copy(rows, acc_s.at[dst_b], add=True)
        return carry
    lax.fori_loop(0, NCHUNK, chunk, 0)

    pltpu.sync_copy(denom_t, den_hbm.at[pl.ds((c * NTILE + s) * NP, NP)])
    plsc.subcore_barrier()
    for k in range(NP // NTILE // CH):
        r0 = base + k * CH
        pltpu.sync_copy(acc_s.at[pl.ds(r0, CH)], out_hbm.at[pl.ds(row_off + r0, CH)])


@functools.cache
def _sc_edge_kernel():
    mesh = plsc.VectorSubcoreMesh(core_axis_name="c", subcore_axis_name="s",
                                  num_cores=NSC, num_subcores=NTILE)
    return pl.kernel(
        _sc_edge_body,
        out_type=(
            jax.ShapeDtypeStruct((H * NP, C), jnp.float32),
            jax.ShapeDtypeStruct((H * NTILE * NP,), jnp.float32),
        ),
        mesh=mesh,
        compiler_params=pltpu.CompilerParams(needs_layout_passes=False,
                                             use_tc_tiling_on_sc=False),
        scratch_types=[
            pltpu.VMEM((NP,), jnp.float32),
            pltpu.VMEM((NP,), jnp.float32),
            pltpu.VMEM((16,), jnp.float32),
            pltpu.VMEM((CH,), jnp.int32),
            pltpu.VMEM((CH,), jnp.int32),
            pltpu.VMEM((CH,), jnp.float32),
            pltpu.VMEM((CH, C), jnp.float32),
            pltpu.VMEM((NP,), jnp.float32),
            pltpu.VMEM_SHARED((NP, C), jnp.float32),
            pltpu.SemaphoreType.DMA,
        ],
    )


def kernel(x, edge_index, W1, as1, ad1, b1, g1, be1, W2, as2, ad2, b2, g2, be2,
           Wh, bh):
    loops = jnp.arange(N, dtype=jnp.int32)
    npad = EPAD - E - N
    src = jnp.concatenate([edge_index[0], loops,
                           jnp.full((npad,), TRASH, jnp.int32)])
    dst = jnp.concatenate([edge_index[1], loops,
                           jnp.full((npad,), TRASH, jnp.int32)])

    def layer(xin, w, a_s, a_d):
        xlp, asv, adv, m = _tc_prep(xin, w, a_s, a_d)
        acc, den = _sc_edge_kernel()(
            xlp.reshape(H * NP, C), asv.reshape(H * NP),
            adv.reshape(H * NP), m.reshape(H * 16), src, dst)
        return acc.reshape(H, NP, C), den.reshape(H, NTILE, NP)

    acc1, den1 = layer(x, W1, as1, ad1)
    h1 = pl.pallas_call(
        _finish_body, out_shape=jax.ShapeDtypeStruct((N, C), jnp.float32),
    )(acc1, den1, x, b1, g1, be1)
    acc2, den2 = layer(h1, W2, as2, ad2)
    y = pl.pallas_call(
        _head_body, out_shape=jax.ShapeDtypeStruct((N,), jnp.float32),
    )(acc2, den2, h1, b2, g2, be2, Wh, bh)
    return y
